# Initial kernel scaffold; baseline (speedup 1.0000x reference)
import jax
import jax.numpy as jnp
from jax.experimental import pallas as pl


def kernel(x, edge_index, edge_attr, pos, W_msg1, b_msg1, W_msg2, b_msg2, W_upd1, b_upd1, W_upd2, b_upd2, W_pos, W_edge, b_edge):
    raise NotImplementedError("write your pallas kernel here")


# speedup vs baseline: 3.1722x; 3.1722x over previous
import dataclasses
import functools

import jax
import jax.numpy as jnp
from jax import lax
from jax.experimental import pallas as pl
from jax.experimental.pallas import tpu as pltpu
from jax.experimental.pallas import tpu_sc as plsc

N = 10000
E = 320000
D = 128
GW = 80
SW = 80
CH = 2000
BE = 1280
BN = 2000
NC = 2
NS = 16
NW = NC * NS
ZR = 624
ZTAIL = N - NS * ZR
CR = 48
NP = 1280


def _silu(v):
    return v * jax.nn.sigmoid(v)


def _proj_body(x_ref, w1a_ref, w1b_ref, b1_ref, ta_ref, tb_ref):
    x = x_ref[...]
    ta_ref[...] = jnp.dot(x, w1a_ref[...], preferred_element_type=jnp.float32)
    tb_ref[...] = (
        jnp.dot(x, w1b_ref[...], preferred_element_type=jnp.float32) + b1_ref[...]
    )


def _proj(x, w1a, w1b, b1r):
    nb = N // BN
    return pl.pallas_call(
        _proj_body,
        grid=(nb,),
        in_specs=[
            pl.BlockSpec((BN, D), lambda i: (i, 0)),
            pl.BlockSpec((D, D), lambda i: (0, 0)),
            pl.BlockSpec((D, D), lambda i: (0, 0)),
            pl.BlockSpec((1, D), lambda i: (0, 0)),
        ],
        out_specs=[
            pl.BlockSpec((BN, D), lambda i: (i, 0)),
            pl.BlockSpec((BN, D), lambda i: (i, 0)),
        ],
        out_shape=[
            jax.ShapeDtypeStruct((N, D), jnp.float32),
            jax.ShapeDtypeStruct((N, D), jnp.float32),
        ],
    )(x, w1a, w1b, b1r)


def _pose(posflat, srcw, dstw):
    mesh = plsc.VectorSubcoreMesh(
        core_axis_name="core", subcore_axis_name="subcore"
    )
    nwin_total = E // CH
    nwin = nwin_total // NW

    cp = pltpu.CompilerParams()
    if "needs_layout_passes" in pltpu.CompilerParams.__dataclass_fields__:
        cp = dataclasses.replace(cp, needs_layout_passes=False)

    @pl.kernel(
        out_type=jax.ShapeDtypeStruct((E * 16,), jnp.float32),
        mesh=mesh,
        compiler_params=cp,
        scratch_types=[
            pltpu.VMEM((N * 4,), jnp.float32),
            pltpu.VMEM((1, CH), jnp.int32),
            pltpu.VMEM((1, CH), jnp.int32),
            pltpu.VMEM((CH * 16,), jnp.float32),
        ],
    )
    def k(pos_hbm, src_hbm, dst_hbm, pose_hbm, ptab, src_v, dst_v, buf):
        c = lax.axis_index("core")
        s = lax.axis_index("subcore")
        wid = c * NS + s
        pltpu.sync_copy(pos_hbm, ptab)

        @pl.loop(0, nwin)
        def _(t):
            w = wid * nwin + t
            pltpu.sync_copy(src_hbm.at[w], src_v)
            pltpu.sync_copy(dst_hbm.at[w], dst_v)

            @pl.loop(0, CH // 16)
            def _(i):
                si = src_v[0, pl.ds(i * 16, 16)] * 4
                di = dst_v[0, pl.ds(i * 16, 16)] * 4
                psx = plsc.load_gather(ptab, [si])
                psy = plsc.load_gather(ptab, [si + 1])
                psz = plsc.load_gather(ptab, [si + 2])
                pdx = plsc.load_gather(ptab, [di])
                pdy = plsc.load_gather(ptab, [di + 1])
                pdz = plsc.load_gather(ptab, [di + 2])
                dx = psx - pdx
                dy = psy - pdy
                dz = psz - pdz
                d2 = dx * dx + dy * dy + dz * dz
                flat0 = lax.iota(jnp.int32, 16) * 16 + i * 256
                plsc.store_scatter(buf, [flat0], psx)
                plsc.store_scatter(buf, [flat0 + 1], psy)
                plsc.store_scatter(buf, [flat0 + 2], psz)
                plsc.store_scatter(buf, [flat0 + 3], d2)

            pltpu.sync_copy(buf, pose_hbm.at[pl.ds(w * CH * 16, CH * 16)])

    return k(posflat, srcw, dstw)


def _gather(table_a, table_b, src3d, dst3d):
    mesh = plsc.VectorSubcoreMesh(
        core_axis_name="core", subcore_axis_name="subcore"
    )

    @pl.kernel(
        out_type=(
            jax.ShapeDtypeStruct((E, D), jnp.float32),
            jax.ShapeDtypeStruct((E, D), jnp.float32),
        ),
        mesh=mesh,
    )
    def k(ta_hbm, tb_hbm, src_hbm, dst_hbm, ga_hbm, gb_hbm):
        def body(si_vmem, di_vmem, ga_vmem, gb_vmem):
            pltpu.sync_copy(ta_hbm.at[si_vmem.at[0, 0]], ga_vmem)
            pltpu.sync_copy(tb_hbm.at[di_vmem.at[0, 0]], gb_vmem)

        pltpu.emit_pipeline(
            body,
            grid=(E // GW,),
            in_specs=[
                pl.BlockSpec((1, 1, GW), lambda i: (i, 0, 0)),
                pl.BlockSpec((1, 1, GW), lambda i: (i, 0, 0)),
            ],
            out_specs=[
                pl.BlockSpec((GW, D), lambda i: (i, 0)),
                pl.BlockSpec((GW, D), lambda i: (i, 0)),
            ],
            core_axis_name=("core", "subcore"),
            dimension_semantics=(pltpu.PARALLEL,),
        )(src_hbm, dst_hbm, ga_hbm, gb_hbm)

    return k(table_a, table_b, src3d, dst3d)


def _edge_body(ga_ref, gb_ref, ea_ref, pose_ref, dm8_ref, w1c_ref, w2_ref,
               we_ref, wp_ref, w1d_ref, b2_ref, be_ref, m_ref, eo_ref, pm_ref):
    ga = ga_ref[...]
    gb = gb_ref[...]
    ea = ea_ref[...]
    pose8 = pose_ref[...]
    d2 = pose8[:, 3:4]
    pre = (
        ga + gb
        + jnp.dot(ea, w1c_ref[...], preferred_element_type=jnp.float32)
        + d2 * w1d_ref[...]
    )
    m1 = _silu(pre)
    m = _silu(jnp.dot(m1, w2_ref[...], preferred_element_type=jnp.float32)
              + b2_ref[...])
    m_ref[...] = m
    eo_ref[...] = ea + _silu(
        jnp.dot(m, we_ref[...], preferred_element_type=jnp.float32) + be_ref[...]
    )
    coef = jnp.tanh(jnp.sum(m * wp_ref[...], axis=1, keepdims=True))
    lane = lax.broadcasted_iota(jnp.int32, (BE, D), 1)
    lane16 = lane % 16
    group = lane // 16
    f32 = jnp.float32
    pmval = (
        pose8[:, 0:1] * coef * (lane16 == 0).astype(f32)
        + pose8[:, 1:2] * coef * (lane16 == 1).astype(f32)
        + pose8[:, 2:3] * coef * (lane16 == 2).astype(f32)
        + coef * (lane16 == 3).astype(f32)
        + (lane16 == 4).astype(f32)
    )
    pm_ref[...] = pmval * (group == dm8_ref[...]).astype(f32)


def _edge(ga, gb, ea, pose, dm8, w1c, w2, we, wp_row, w1d_row, b2r, ber):
    nb = E // BE
    full = lambda r, c: pl.BlockSpec((r, c), lambda i: (0, 0))
    return pl.pallas_call(
        _edge_body,
        grid=(nb,),
        in_specs=[
            pl.BlockSpec((BE, D), lambda i: (i, 0)),
            pl.BlockSpec((BE, D), lambda i: (i, 0)),
            pl.BlockSpec((BE, D), lambda i: (i, 0)),
            pl.BlockSpec((BE, 16), lambda i: (i, 0)),
            pl.BlockSpec((BE, 1), lambda i: (i, 0)),
            full(D, D), full(D, D), full(D, D),
            full(1, D), full(1, D), full(1, D), full(1, D),
        ],
        out_specs=[
            pl.BlockSpec((BE, D), lambda i: (i, 0)),
            pl.BlockSpec((BE, D), lambda i: (i, 0)),
            pl.BlockSpec((BE, D), lambda i: (i, 0)),
        ],
        out_shape=[
            jax.ShapeDtypeStruct((E, D), jnp.float32),
            jax.ShapeDtypeStruct((E, D), jnp.float32),
            jax.ShapeDtypeStruct((E, D), jnp.float32),
        ],
    )(ga, gb, ea, pose, dm8, w1c, w2, we, wp_row, w1d_row, b2r, ber)


def _scatter(m, pm128, dst3d, dst8_3d, zrows):
    mesh = plsc.VectorSubcoreMesh(
        core_axis_name="core", subcore_axis_name="subcore"
    )
    chunk = E // NW
    nwin = chunk // SW
    pr = NP // NS

    @pl.kernel(
        out_type=(
            jax.ShapeDtypeStruct((NC * N, D), jnp.float32),
            jax.ShapeDtypeStruct((NC * NP, D), jnp.float32),
        ),
        mesh=mesh,
        scratch_types=[
            pltpu.VMEM_SHARED((N, D), jnp.float32),
            pltpu.VMEM_SHARED((NP, D), jnp.float32),
            pltpu.VMEM((1, SW), jnp.int32),
            pltpu.VMEM((1, SW), jnp.int32),
            pltpu.VMEM((SW, D), jnp.float32),
            pltpu.VMEM((SW, D), jnp.float32),
            pltpu.VMEM((CR, D), jnp.float32),
        ],
    )
    def k(m_hbm, pm_hbm, dst_hbm, dst8_hbm, z_hbm, aggp_hbm, posp_hbm,
          acc, accp, idx_v, idx8_v, m_v, pm_v, bb):
        c = lax.axis_index("core")
        s = lax.axis_index("subcore")
        pltpu.sync_copy(z_hbm, bb)

        @pl.loop(0, ZR // CR)
        def _(t):
            pltpu.sync_copy(bb, acc.at[pl.ds(s * ZR + t * CR, CR)])

        pltpu.sync_copy(bb, accp.at[pl.ds(s * pr, CR)])
        pltpu.sync_copy(bb.at[pl.ds(0, pr - CR)],
                        accp.at[pl.ds(s * pr + CR, pr - CR)])

        @pl.when(s == 0)
        def _():
            pltpu.sync_copy(bb.at[pl.ds(0, ZTAIL)],
                            acc.at[pl.ds(NS * ZR, ZTAIL)])

        plsc.subcore_barrier()

        wbase = (c * NS + s) * nwin

        @pl.loop(0, nwin)
        def _(j):
            off = (wbase + j) * SW
            pltpu.sync_copy(dst_hbm.at[wbase + j], idx_v)
            pltpu.sync_copy(dst8_hbm.at[wbase + j], idx8_v)
            pltpu.sync_copy(m_hbm.at[pl.ds(off, SW)], m_v)
            pltpu.sync_copy(pm_hbm.at[pl.ds(off, SW)], pm_v)
            pltpu.sync_copy(m_v, acc.at[idx_v.at[0]], add=True)
            pltpu.sync_copy(pm_v, accp.at[idx8_v.at[0]], add=True)

        plsc.subcore_barrier()

        @pl.loop(0, ZR // CR)
        def _(t):
            r0 = s * ZR + t * CR
            pltpu.sync_copy(acc.at[pl.ds(r0, CR)], bb)
            pltpu.sync_copy(bb, aggp_hbm.at[pl.ds(c * N + r0, CR)])

        pltpu.sync_copy(accp.at[pl.ds(s * pr, CR)], bb)
        pltpu.sync_copy(bb, posp_hbm.at[pl.ds(c * NP + s * pr, CR)])
        pltpu.sync_copy(accp.at[pl.ds(s * pr + CR, pr - CR)],
                        bb.at[pl.ds(0, pr - CR)])
        pltpu.sync_copy(bb.at[pl.ds(0, pr - CR)],
                        posp_hbm.at[pl.ds(c * NP + s * pr + CR, pr - CR)])

        @pl.when(s == 0)
        def _():
            pltpu.sync_copy(acc.at[pl.ds(NS * ZR, ZTAIL)], bb.at[pl.ds(0, ZTAIL)])
            pltpu.sync_copy(bb.at[pl.ds(0, ZTAIL)],
                            aggp_hbm.at[pl.ds(c * N + NS * ZR, ZTAIL)])

    return k(m, pm128, dst3d, dst8_3d, zrows)


def _node_body(x_ref, pos_ref, aggp_ref, posp_ref, wu1a_ref, wu1b_ref,
               wu2_ref, bu1_ref, bu2_ref, node_ref, pos_out_ref):
    x = x_ref[...]
    agg = aggp_ref[0] + aggp_ref[1]
    h = _silu(
        jnp.dot(x, wu1a_ref[...], preferred_element_type=jnp.float32)
        + jnp.dot(agg, wu1b_ref[...], preferred_element_type=jnp.float32)
        + bu1_ref[...]
    )
    node_ref[...] = (
        x + jnp.dot(h, wu2_ref[...], preferred_element_type=jnp.float32)
        + bu2_ref[...]
    )
    pos = pos_ref[...]
    pp = posp_ref[0] + posp_ref[1]
    cnt = pp[:, 4:5]
    pos_agg = pp[:, :3] - pos * pp[:, 3:4]
    pos_out_ref[...] = pos + pos_agg / jnp.maximum(cnt, 1.0)


def _node(x, pos, aggp, posp, wu1a, wu1b, wu2, bu1r, bu2r):
    nb = N // BN
    full = lambda r, c: pl.BlockSpec((r, c), lambda i: (0, 0))
    return pl.pallas_call(
        _node_body,
        grid=(nb,),
        in_specs=[
            pl.BlockSpec((BN, D), lambda i: (i, 0)),
            pl.BlockSpec((BN, 3), lambda i: (i, 0)),
            pl.BlockSpec((NC, BN, D), lambda i: (0, i, 0)),
            pl.BlockSpec((NC, BN, 16), lambda i: (0, i, 0)),
            full(D, D), full(D, D), full(D, D), full(1, D), full(1, D),
        ],
        out_specs=[
            pl.BlockSpec((BN, D), lambda i: (i, 0)),
            pl.BlockSpec((BN, 3), lambda i: (i, 0)),
        ],
        out_shape=[
            jax.ShapeDtypeStruct((N, D), jnp.float32),
            jax.ShapeDtypeStruct((N, 3), jnp.float32),
        ],
    )(x, pos, aggp, posp, wu1a, wu1b, wu2, bu1r, bu2r)


def kernel(x, edge_index, edge_attr, pos, W_msg1, b_msg1, W_msg2, b_msg2,
           W_upd1, b_upd1, W_upd2, b_upd2, W_pos, W_edge, b_edge):
    src = edge_index[0].astype(jnp.int32)
    dst = edge_index[1].astype(jnp.int32)
    src3d = src.reshape(E // GW, 1, GW)
    dst3d = dst.reshape(E // SW, 1, SW)
    srcw = src.reshape(E // CH, 1, CH)
    dstw = dst.reshape(E // CH, 1, CH)
    posflat = jnp.pad(pos, ((0, 0), (0, 1))).reshape(-1)

    w1a = W_msg1[:D]
    w1b = W_msg1[D:2 * D]
    w1c = W_msg1[2 * D:3 * D]
    w1d_row = W_msg1[3 * D:3 * D + 1]
    b1r = b_msg1.reshape(1, D)
    b2r = b_msg2.reshape(1, D)
    ber = b_edge.reshape(1, D)
    wp_row = W_pos.reshape(1, D)
    wu1a = W_upd1[:D]
    wu1b = W_upd1[D:]
    bu1r = b_upd1.reshape(1, D)
    bu2r = b_upd2.reshape(1, D)

    pose = _pose(posflat, srcw, dstw).reshape(E, 16)
    ta, tb = _proj(x, w1a, w1b, b1r)
    ga, gb = _gather(ta, tb, src3d, dst3d)
    dm8 = (dst % 8).reshape(E, 1)
    dst8_3d = (dst // 8).reshape(E // SW, 1, SW)
    m, edge_out, pm128 = _edge(ga, gb, edge_attr, pose, dm8, w1c, W_msg2,
                               W_edge, wp_row, w1d_row, b2r, ber)
    zrows = jnp.zeros((CR, D), jnp.float32)
    aggp, posp = _scatter(m, pm128, dst3d, dst8_3d, zrows)
    aggp = aggp.reshape(NC, N, D)
    posp = posp.reshape(NC, NP * 8, 16)[:, :N]
    node_out, pos_out = _node(x, pos, aggp, posp, wu1a, wu1b, W_upd2,
                              bu1r, bu2r)
    return (node_out, edge_out, pos_out)


# speedup vs baseline: 3.7488x; 1.1817x over previous
import dataclasses
import functools

import jax
import jax.numpy as jnp
from jax import lax
from jax.experimental import pallas as pl
from jax.experimental.pallas import tpu as pltpu
from jax.experimental.pallas import tpu_sc as plsc

N = 10000
E = 320000
D = 128
GW = 80
SW = 40
CH = 2000
BE = 1280
BN = 2000
NC = 2
NS = 16
NW = NC * NS
ZR = 624
ZTAIL = N - NS * ZR
CR = 48
NP = 1280


def _silu(v):
    return v * jax.nn.sigmoid(v)


def _proj_body(x_ref, w1a_ref, w1b_ref, b1_ref, ta_ref, tb_ref):
    x = x_ref[...]
    ta_ref[...] = jnp.dot(x, w1a_ref[...], preferred_element_type=jnp.float32)
    tb_ref[...] = (
        jnp.dot(x, w1b_ref[...], preferred_element_type=jnp.float32) + b1_ref[...]
    )


def _proj(x, w1a, w1b, b1r):
    nb = N // BN
    return pl.pallas_call(
        _proj_body,
        grid=(nb,),
        in_specs=[
            pl.BlockSpec((BN, D), lambda i: (i, 0)),
            pl.BlockSpec((D, D), lambda i: (0, 0)),
            pl.BlockSpec((D, D), lambda i: (0, 0)),
            pl.BlockSpec((1, D), lambda i: (0, 0)),
        ],
        out_specs=[
            pl.BlockSpec((BN, D), lambda i: (i, 0)),
            pl.BlockSpec((BN, D), lambda i: (i, 0)),
        ],
        out_shape=[
            jax.ShapeDtypeStruct((N, D), jnp.float32),
            jax.ShapeDtypeStruct((N, D), jnp.float32),
        ],
    )(x, w1a, w1b, b1r)


def _pose(posflat, srcw, dstw):
    mesh = plsc.VectorSubcoreMesh(
        core_axis_name="core", subcore_axis_name="subcore"
    )
    nwin_total = E // CH
    nwin = nwin_total // NW

    cp = pltpu.CompilerParams()
    if "needs_layout_passes" in pltpu.CompilerParams.__dataclass_fields__:
        cp = dataclasses.replace(cp, needs_layout_passes=False)

    @pl.kernel(
        out_type=jax.ShapeDtypeStruct((E * 16,), jnp.float32),
        mesh=mesh,
        compiler_params=cp,
        scratch_types=[
            pltpu.VMEM((N * 4,), jnp.float32),
            pltpu.VMEM((1, CH), jnp.int32),
            pltpu.VMEM((1, CH), jnp.int32),
            pltpu.VMEM((CH * 16,), jnp.float32),
        ],
    )
    def k(pos_hbm, src_hbm, dst_hbm, pose_hbm, ptab, src_v, dst_v, buf):
        c = lax.axis_index("core")
        s = lax.axis_index("subcore")
        wid = c * NS + s
        pltpu.sync_copy(pos_hbm, ptab)

        @pl.loop(0, nwin)
        def _(t):
            w = wid * nwin + t
            pltpu.sync_copy(src_hbm.at[w], src_v)
            pltpu.sync_copy(dst_hbm.at[w], dst_v)

            @pl.loop(0, CH // 16)
            def _(i):
                si = src_v[0, pl.ds(i * 16, 16)] * 4
                di = dst_v[0, pl.ds(i * 16, 16)] * 4
                psx = plsc.load_gather(ptab, [si])
                psy = plsc.load_gather(ptab, [si + 1])
                psz = plsc.load_gather(ptab, [si + 2])
                pdx = plsc.load_gather(ptab, [di])
                pdy = plsc.load_gather(ptab, [di + 1])
                pdz = plsc.load_gather(ptab, [di + 2])
                dx = psx - pdx
                dy = psy - pdy
                dz = psz - pdz
                d2 = dx * dx + dy * dy + dz * dz
                flat0 = lax.iota(jnp.int32, 16) * 16 + i * 256
                plsc.store_scatter(buf, [flat0], psx)
                plsc.store_scatter(buf, [flat0 + 1], psy)
                plsc.store_scatter(buf, [flat0 + 2], psz)
                plsc.store_scatter(buf, [flat0 + 3], d2)

            pltpu.sync_copy(buf, pose_hbm.at[pl.ds(w * CH * 16, CH * 16)])

    return k(posflat, srcw, dstw)


def _gather(table_a, table_b, src3d, dst3d):
    mesh = plsc.VectorSubcoreMesh(
        core_axis_name="core", subcore_axis_name="subcore"
    )

    @pl.kernel(
        out_type=(
            jax.ShapeDtypeStruct((E, D), jnp.float32),
            jax.ShapeDtypeStruct((E, D), jnp.float32),
        ),
        mesh=mesh,
    )
    def k(ta_hbm, tb_hbm, src_hbm, dst_hbm, ga_hbm, gb_hbm):
        def body(si_vmem, di_vmem, ga_vmem, gb_vmem):
            pltpu.sync_copy(ta_hbm.at[si_vmem.at[0, 0]], ga_vmem)
            pltpu.sync_copy(tb_hbm.at[di_vmem.at[0, 0]], gb_vmem)

        pltpu.emit_pipeline(
            body,
            grid=(E // GW,),
            in_specs=[
                pl.BlockSpec((1, 1, GW), lambda i: (i, 0, 0)),
                pl.BlockSpec((1, 1, GW), lambda i: (i, 0, 0)),
            ],
            out_specs=[
                pl.BlockSpec((GW, D), lambda i: (i, 0)),
                pl.BlockSpec((GW, D), lambda i: (i, 0)),
            ],
            core_axis_name=("core", "subcore"),
            dimension_semantics=(pltpu.PARALLEL,),
        )(src_hbm, dst_hbm, ga_hbm, gb_hbm)

    return k(table_a, table_b, src3d, dst3d)


def _edge_body(ga_ref, gb_ref, ea_ref, pose_ref, dm8_ref, w1c_ref, w2_ref,
               we_ref, wp_ref, w1d_ref, b2_ref, be_ref, m_ref, eo_ref, pm_ref):
    bf16 = jnp.bfloat16
    ga = ga_ref[...].astype(jnp.float32)
    gb = gb_ref[...].astype(jnp.float32)
    ea = ea_ref[...]
    pose8 = pose_ref[...]
    d2 = pose8[:, 3:4]
    pre = (
        ga + gb
        + jnp.dot(ea.astype(bf16), w1c_ref[...],
                  preferred_element_type=jnp.float32)
        + d2 * w1d_ref[...]
    )
    m1 = _silu(pre)
    m = _silu(jnp.dot(m1.astype(bf16), w2_ref[...],
                      preferred_element_type=jnp.float32) + b2_ref[...])
    m_ref[...] = m
    eo_ref[...] = ea + _silu(
        jnp.dot(m.astype(bf16), we_ref[...],
                preferred_element_type=jnp.float32) + be_ref[...]
    )
    coef = jnp.tanh(jnp.sum(m * wp_ref[...], axis=1, keepdims=True))
    lane = lax.broadcasted_iota(jnp.int32, (BE, D), 1)
    lane16 = lane % 16
    group = lane // 16
    f32 = jnp.float32
    pmval = (
        pose8[:, 0:1] * coef * (lane16 == 0).astype(f32)
        + pose8[:, 1:2] * coef * (lane16 == 1).astype(f32)
        + pose8[:, 2:3] * coef * (lane16 == 2).astype(f32)
        + coef * (lane16 == 3).astype(f32)
        + (lane16 == 4).astype(f32)
    )
    pm_ref[...] = pmval * (group == dm8_ref[...]).astype(f32)


def _edge(ga, gb, ea, pose, dm8, w1c, w2, we, wp_row, w1d_row, b2r, ber):
    nb = E // BE
    full = lambda r, c: pl.BlockSpec((r, c), lambda i: (0, 0))
    return pl.pallas_call(
        _edge_body,
        grid=(nb,),
        in_specs=[
            pl.BlockSpec((BE, D), lambda i: (i, 0)),
            pl.BlockSpec((BE, D), lambda i: (i, 0)),
            pl.BlockSpec((BE, D), lambda i: (i, 0)),
            pl.BlockSpec((BE, 16), lambda i: (i, 0)),
            pl.BlockSpec((BE, 1), lambda i: (i, 0)),
            full(D, D), full(D, D), full(D, D),
            full(1, D), full(1, D), full(1, D), full(1, D),
        ],
        out_specs=[
            pl.BlockSpec((BE, D), lambda i: (i, 0)),
            pl.BlockSpec((BE, D), lambda i: (i, 0)),
            pl.BlockSpec((BE, D), lambda i: (i, 0)),
        ],
        out_shape=[
            jax.ShapeDtypeStruct((E, D), jnp.float32),
            jax.ShapeDtypeStruct((E, D), jnp.float32),
            jax.ShapeDtypeStruct((E, D), jnp.float32),
        ],
    )(ga, gb, ea, pose, dm8, w1c, w2, we, wp_row, w1d_row, b2r, ber)


def _scatter(m, pm128, dst3d, dst8_3d, zrows):
    mesh = plsc.VectorSubcoreMesh(
        core_axis_name="core", subcore_axis_name="subcore"
    )
    chunk = E // NW
    nwin = chunk // SW
    pr = NP // NS

    @pl.kernel(
        out_type=(
            jax.ShapeDtypeStruct((NC * N, D), jnp.float32),
            jax.ShapeDtypeStruct((NC * NP, D), jnp.float32),
        ),
        mesh=mesh,
        scratch_types=[
            pltpu.VMEM_SHARED((N, D), jnp.float32),
            pltpu.VMEM_SHARED((NP, D), jnp.float32),
            pltpu.VMEM((1, SW), jnp.int32),
            pltpu.VMEM((1, SW), jnp.int32),
            pltpu.VMEM((SW, D), jnp.float32),
            pltpu.VMEM((SW, D), jnp.float32),
            pltpu.VMEM((1, SW), jnp.int32),
            pltpu.VMEM((1, SW), jnp.int32),
            pltpu.VMEM((SW, D), jnp.float32),
            pltpu.VMEM((SW, D), jnp.float32),
            pltpu.VMEM((CR, D), jnp.float32),
            pltpu.SemaphoreType.DMA,
            pltpu.SemaphoreType.DMA,
        ],
    )
    def k(m_hbm, pm_hbm, dst_hbm, dst8_hbm, z_hbm, aggp_hbm, posp_hbm,
          acc, accp, idx_v0, idx8_v0, m_v0, pm_v0,
          idx_v1, idx8_v1, m_v1, pm_v1, bb, sem0, sem1):
        c = lax.axis_index("core")
        s = lax.axis_index("subcore")
        pltpu.sync_copy(z_hbm, bb)

        @pl.loop(0, ZR // CR)
        def _(t):
            pltpu.sync_copy(bb, acc.at[pl.ds(s * ZR + t * CR, CR)])

        pltpu.sync_copy(bb, accp.at[pl.ds(s * pr, CR)])
        pltpu.sync_copy(bb.at[pl.ds(0, pr - CR)],
                        accp.at[pl.ds(s * pr + CR, pr - CR)])

        @pl.when(s == 0)
        def _():
            pltpu.sync_copy(bb.at[pl.ds(0, ZTAIL)],
                            acc.at[pl.ds(NS * ZR, ZTAIL)])

        plsc.subcore_barrier()

        wbase = (c * NS + s) * nwin

        def loads(w, idxb, idx8b, mb, pmb, sem):
            off = w * SW
            pltpu.async_copy(dst_hbm.at[w], idxb, sem)
            pltpu.async_copy(dst8_hbm.at[w], idx8b, sem)
            pltpu.async_copy(m_hbm.at[pl.ds(off, SW)], mb, sem)
            pltpu.async_copy(pm_hbm.at[pl.ds(off, SW)], pmb, sem)

        def waits(w, idxb, idx8b, mb, pmb, sem):
            off = w * SW
            pltpu.make_async_copy(dst_hbm.at[w], idxb, sem).wait()
            pltpu.make_async_copy(dst8_hbm.at[w], idx8b, sem).wait()
            pltpu.make_async_copy(m_hbm.at[pl.ds(off, SW)], mb, sem).wait()
            pltpu.make_async_copy(pm_hbm.at[pl.ds(off, SW)], pmb, sem).wait()

        def adds(idxb, idx8b, mb, pmb):
            pltpu.sync_copy(mb, acc.at[idxb.at[0]], add=True)
            pltpu.sync_copy(pmb, accp.at[idx8b.at[0]], add=True)

        loads(wbase, idx_v0, idx8_v0, m_v0, pm_v0, sem0)

        @pl.loop(0, nwin, step=2)
        def _(j):
            w0 = wbase + j
            waits(w0, idx_v0, idx8_v0, m_v0, pm_v0, sem0)
            loads(w0 + 1, idx_v1, idx8_v1, m_v1, pm_v1, sem1)
            adds(idx_v0, idx8_v0, m_v0, pm_v0)
            waits(w0 + 1, idx_v1, idx8_v1, m_v1, pm_v1, sem1)

            @pl.when(j + 2 < nwin)
            def _():
                loads(w0 + 2, idx_v0, idx8_v0, m_v0, pm_v0, sem0)

            adds(idx_v1, idx8_v1, m_v1, pm_v1)

        plsc.subcore_barrier()

        @pl.loop(0, ZR // CR)
        def _(t):
            r0 = s * ZR + t * CR
            pltpu.sync_copy(acc.at[pl.ds(r0, CR)], bb)
            pltpu.sync_copy(bb, aggp_hbm.at[pl.ds(c * N + r0, CR)])

        pltpu.sync_copy(accp.at[pl.ds(s * pr, CR)], bb)
        pltpu.sync_copy(bb, posp_hbm.at[pl.ds(c * NP + s * pr, CR)])
        pltpu.sync_copy(accp.at[pl.ds(s * pr + CR, pr - CR)],
                        bb.at[pl.ds(0, pr - CR)])
        pltpu.sync_copy(bb.at[pl.ds(0, pr - CR)],
                        posp_hbm.at[pl.ds(c * NP + s * pr + CR, pr - CR)])

        @pl.when(s == 0)
        def _():
            pltpu.sync_copy(acc.at[pl.ds(NS * ZR, ZTAIL)], bb.at[pl.ds(0, ZTAIL)])
            pltpu.sync_copy(bb.at[pl.ds(0, ZTAIL)],
                            aggp_hbm.at[pl.ds(c * N + NS * ZR, ZTAIL)])

    return k(m, pm128, dst3d, dst8_3d, zrows)


def _node_body(x_ref, pos_ref, aggp_ref, posp_ref, wu1a_ref, wu1b_ref,
               wu2_ref, bu1_ref, bu2_ref, node_ref, pos_out_ref):
    x = x_ref[...]
    agg = aggp_ref[0] + aggp_ref[1]
    h = _silu(
        jnp.dot(x, wu1a_ref[...], preferred_element_type=jnp.float32)
        + jnp.dot(agg, wu1b_ref[...], preferred_element_type=jnp.float32)
        + bu1_ref[...]
    )
    node_ref[...] = (
        x + jnp.dot(h, wu2_ref[...], preferred_element_type=jnp.float32)
        + bu2_ref[...]
    )
    pos = pos_ref[...]
    pp = posp_ref[0] + posp_ref[1]
    cnt = pp[:, 4:5]
    pos_agg = pp[:, :3] - pos * pp[:, 3:4]
    pos_out_ref[...] = pos + pos_agg / jnp.maximum(cnt, 1.0)


def _node(x, pos, aggp, posp, wu1a, wu1b, wu2, bu1r, bu2r):
    nb = N // BN
    full = lambda r, c: pl.BlockSpec((r, c), lambda i: (0, 0))
    return pl.pallas_call(
        _node_body,
        grid=(nb,),
        in_specs=[
            pl.BlockSpec((BN, D), lambda i: (i, 0)),
            pl.BlockSpec((BN, 3), lambda i: (i, 0)),
            pl.BlockSpec((NC, BN, D), lambda i: (0, i, 0)),
            pl.BlockSpec((NC, BN, 16), lambda i: (0, i, 0)),
            full(D, D), full(D, D), full(D, D), full(1, D), full(1, D),
        ],
        out_specs=[
            pl.BlockSpec((BN, D), lambda i: (i, 0)),
            pl.BlockSpec((BN, 3), lambda i: (i, 0)),
        ],
        out_shape=[
            jax.ShapeDtypeStruct((N, D), jnp.float32),
            jax.ShapeDtypeStruct((N, 3), jnp.float32),
        ],
    )(x, pos, aggp, posp, wu1a, wu1b, wu2, bu1r, bu2r)


def kernel(x, edge_index, edge_attr, pos, W_msg1, b_msg1, W_msg2, b_msg2,
           W_upd1, b_upd1, W_upd2, b_upd2, W_pos, W_edge, b_edge):
    src = edge_index[0].astype(jnp.int32)
    dst = edge_index[1].astype(jnp.int32)
    src3d = src.reshape(E // GW, 1, GW)
    dstg3d = dst.reshape(E // GW, 1, GW)
    dst3d = dst.reshape(E // SW, 1, SW)
    srcw = src.reshape(E // CH, 1, CH)
    dstw = dst.reshape(E // CH, 1, CH)
    posflat = jnp.pad(pos, ((0, 0), (0, 1))).reshape(-1)

    w1a = W_msg1[:D]
    w1b = W_msg1[D:2 * D]
    w1c = W_msg1[2 * D:3 * D]
    w1d_row = W_msg1[3 * D:3 * D + 1]
    b1r = b_msg1.reshape(1, D)
    b2r = b_msg2.reshape(1, D)
    ber = b_edge.reshape(1, D)
    wp_row = W_pos.reshape(1, D)
    wu1a = W_upd1[:D]
    wu1b = W_upd1[D:]
    bu1r = b_upd1.reshape(1, D)
    bu2r = b_upd2.reshape(1, D)

    pose = _pose(posflat, srcw, dstw).reshape(E, 16)
    ta, tb = _proj(x, w1a, w1b, b1r)
    ga, gb = _gather(ta, tb, src3d, dstg3d)
    dm8 = (dst % 8).reshape(E, 1)
    dst8_3d = (dst // 8).reshape(E // SW, 1, SW)
    m, edge_out, pm128 = _edge(ga, gb, edge_attr, pose, dm8,
                               w1c.astype(jnp.bfloat16),
                               W_msg2.astype(jnp.bfloat16),
                               W_edge.astype(jnp.bfloat16),
                               wp_row, w1d_row, b2r, ber)
    zrows = jnp.zeros((CR, D), jnp.float32)
    aggp, posp = _scatter(m, pm128, dst3d, dst8_3d, zrows)
    aggp = aggp.reshape(NC, N, D)
    posp = posp.reshape(NC, NP * 8, 16)[:, :N]
    node_out, pos_out = _node(x, pos, aggp, posp, wu1a, wu1b, W_upd2,
                              bu1r, bu2r)
    return (node_out, edge_out, pos_out)
